# baseline (speedup 1.0000x reference)
import functools

import jax
import jax.numpy as jnp
from jax import lax
from jax.experimental import pallas as pl
from jax.experimental.pallas import tpu as pltpu
from jax.experimental.pallas import tpu_sc as plsc

LAMDA = 0.001

NC = 2
NS = 16
NW = NC * NS
L = 16

B = 16384
D = 64
HW = D // 2
BPW = B // NW
CH = 128
NCHUNK = BPW // CH

N_LOOK = 9


def _sc_body(u_h, i_h, pu_h, pi_h, nu_h, ni_h, r_h, nr_h, wu, wi, wr,
             xhat_o, sq_o,
             u_v, i_v, pu_v, pi_v, nu_v, ni_v, r_v, nr_v,
             idx2_v, bufs,
             xhat_v, squ, sqi, sqpu, sqpi, sqnu, sqni, sqr, sqnr, sem):
    user_num = wu.shape[0] // 2
    wid = lax.axis_index("s") * NC + lax.axis_index("c")
    base = pl.multiple_of(wid * BPW, BPW)

    pltpu.sync_copy(u_h.at[pl.ds(base, BPW)], u_v)
    pltpu.sync_copy(i_h.at[pl.ds(base, BPW)], i_v)
    pltpu.sync_copy(pu_h.at[pl.ds(base, BPW)], pu_v)
    pltpu.sync_copy(pi_h.at[pl.ds(base, BPW)], pi_v)
    pltpu.sync_copy(nu_h.at[pl.ds(base, BPW)], nu_v)
    pltpu.sync_copy(ni_h.at[pl.ds(base, BPW)], ni_v)
    pltpu.sync_copy(r_h.at[pl.ds(base, BPW)], r_v)
    pltpu.sync_copy(nr_h.at[pl.ds(base, BPW)], nr_v)

    def idx_body(k, _):
        s = pl.ds(pl.multiple_of(k * L, L), L)
        rv = r_v[s]
        src = [u_v[s], i_v[s], pu_v[s], pi_v[s], nu_v[s], ni_v[s],
               u_v[s] + rv * user_num,
               pu_v[s] + rv * user_num,
               nu_v[s] + nr_v[s] * user_num]
        for q in range(N_LOOK):
            dq = src[q] * 2
            idx2_v[2 * q, s] = dq
            idx2_v[2 * q + 1, s] = dq + 1
        return 0

    lax.fori_loop(0, BPW // L, idx_body, 0)

    rows0 = lax.iota(jnp.int32, L)
    zero = jnp.zeros((L,), jnp.float32)

    def hsum(v):
        for sh in (8, 4, 2, 1):
            perm = jnp.bitwise_xor(rows0, sh)
            v = v + jnp.take_along_axis(v, perm, axis=0,
                                        mode="promise_in_bounds")
        return v

    tables = [wu, wi, wu, wi, wu, wi, wr, wr, wr]

    for c in range(NCHUNK):
        s = pl.ds(c * CH, CH)
        cps = []
        for q in range(N_LOOK):
            for h in range(2):
                cps.append(pltpu.async_copy(
                    tables[q].at[idx2_v.at[2 * q + h, s]],
                    bufs[2 * q + h], sem))
        for cp in cps:
            cp.wait()

        def group_body(g, _, c=c):
            def row_body(rr, acc):
                xh, au, ai, apu, api, anu, ani, ar, anr = acc
                r = g * L + rr
                part = [zero] * 9
                for k in range(D // L):
                    h = k // 2
                    sk = pl.ds((k % 2) * L, L)
                    uv = bufs[0 + h][r, sk]
                    iv = bufs[2 + h][r, sk]
                    puv = bufs[4 + h][r, sk]
                    piv = bufs[6 + h][r, sk]
                    nuv = bufs[8 + h][r, sk]
                    niv = bufs[10 + h][r, sk]
                    rv = bufs[12 + h][r, sk]
                    prv = bufs[14 + h][r, sk]
                    nrv = bufs[16 + h][r, sk]
                    gv = uv + rv + iv
                    gp = puv + prv + piv
                    gn = nuv + nrv + niv
                    part = [part[0] + gv * (gp - gn), part[1] + uv * uv,
                            part[2] + iv * iv, part[3] + puv * puv,
                            part[4] + piv * piv, part[5] + nuv * nuv,
                            part[6] + niv * niv, part[7] + rv * rv,
                            part[8] + nrv * nrv]
                lane = rows0 == rr
                xh = jnp.where(lane, hsum(part[0]), xh)
                au = jnp.where(lane, hsum(part[1]), au)
                ai = jnp.where(lane, hsum(part[2]), ai)
                apu = jnp.where(lane, hsum(part[3]), apu)
                api = jnp.where(lane, hsum(part[4]), api)
                anu = jnp.where(lane, hsum(part[5]), anu)
                ani = jnp.where(lane, hsum(part[6]), ani)
                ar = jnp.where(lane, hsum(part[7]), ar)
                anr = jnp.where(lane, hsum(part[8]), anr)
                return (xh, au, ai, apu, api, anu, ani, ar, anr)

            xh, au, ai, apu, api, anu, ani, ar, anr = lax.fori_loop(
                0, L, row_body, (zero,) * 9)
            so = pl.ds(pl.multiple_of(c * CH + g * L, L), L)
            xhat_v[so] = xh
            squ[so] = au
            sqi[so] = ai
            sqpu[so] = apu
            sqpi[so] = api
            sqnu[so] = anu
            sqni[so] = ani
            sqr[so] = ar
            sqnr[so] = anr
            return 0

        lax.fori_loop(0, CH // L, group_body, 0)

    pltpu.sync_copy(xhat_v, xhat_o.at[pl.ds(base, BPW)])
    pltpu.sync_copy(squ, sq_o.at[0, pl.ds(base, BPW)])
    pltpu.sync_copy(sqi, sq_o.at[1, pl.ds(base, BPW)])
    pltpu.sync_copy(sqpu, sq_o.at[2, pl.ds(base, BPW)])
    pltpu.sync_copy(sqpi, sq_o.at[3, pl.ds(base, BPW)])
    pltpu.sync_copy(sqnu, sq_o.at[4, pl.ds(base, BPW)])
    pltpu.sync_copy(sqni, sq_o.at[5, pl.ds(base, BPW)])
    pltpu.sync_copy(sqr, sq_o.at[6, pl.ds(base, BPW)])
    pltpu.sync_copy(sqnr, sq_o.at[7, pl.ds(base, BPW)])


def _sc_entry(*args):
    io = args[:13]
    (u_v, i_v, pu_v, pi_v, nu_v, ni_v, r_v, nr_v, idx2_v) = args[13:22]
    bufs = list(args[22:40])
    (xhat_v, squ, sqi, sqpu, sqpi, sqnu, sqni, sqr, sqnr, sem) = args[40:]
    return _sc_body(*io, u_v, i_v, pu_v, pi_v, nu_v, ni_v, r_v, nr_v,
                    idx2_v, bufs, xhat_v, squ, sqi, sqpu, sqpi, sqnu, sqni,
                    sqr, sqnr, sem)


_sc_call = functools.partial(
    pl.kernel,
    out_type=(jax.ShapeDtypeStruct((B,), jnp.float32),
              jax.ShapeDtypeStruct((8, B), jnp.float32)),
    mesh=plsc.VectorSubcoreMesh(core_axis_name="c", subcore_axis_name="s",
                                num_cores=NC, num_subcores=NS),
    scratch_types=(
        [pltpu.VMEM((BPW,), jnp.int32)] * 8
        + [pltpu.VMEM((2 * N_LOOK, BPW), jnp.int32)]
        + [pltpu.VMEM((CH, HW), jnp.float32)] * (2 * N_LOOK)
        + [pltpu.VMEM((BPW,), jnp.float32)] * 9
        + [pltpu.SemaphoreType.DMA]
    ),
    compiler_params=pltpu.CompilerParams(use_tc_tiling_on_sc=False),
)(_sc_entry)


def _fin_body(x_ref, s_ref, loss_ref, reg_ref):
    x = x_ref[...]
    p = 1.0 / (1.0 + jnp.exp(-x))
    loss_ref[0, 0] = -jnp.sum(jnp.log(p))
    reg_ref[0, 0] = jnp.sum(jnp.sqrt(s_ref[...])) * LAMDA


_fin_call = pl.pallas_call(
    _fin_body,
    out_shape=(jax.ShapeDtypeStruct((1, 1), jnp.float32),
               jax.ShapeDtypeStruct((1, 1), jnp.float32)),
    out_specs=(pl.BlockSpec(memory_space=pltpu.SMEM),
               pl.BlockSpec(memory_space=pltpu.SMEM)),
)


def kernel(user_idx, item_idx, pos_user_idx, pos_item_idx, neg_user_idx,
           neg_item_idx, rel_idx, neg_rel_idx, W_user, W_item, W_rel):
    xhat, sq = _sc_call(user_idx.astype(jnp.int32), item_idx.astype(jnp.int32),
                        pos_user_idx.astype(jnp.int32),
                        pos_item_idx.astype(jnp.int32),
                        neg_user_idx.astype(jnp.int32),
                        neg_item_idx.astype(jnp.int32),
                        rel_idx.astype(jnp.int32),
                        neg_rel_idx.astype(jnp.int32),
                        W_user.reshape(-1, HW), W_item.reshape(-1, HW),
                        W_rel.reshape(-1, HW))
    loss, reg = _fin_call(xhat.reshape(128, 128), sq.reshape(1024, 128))
    return (loss[0, 0], reg[0, 0])


# speedup vs baseline: 1.0437x; 1.0437x over previous
import functools

import jax
import jax.numpy as jnp
from jax import lax
from jax.experimental import pallas as pl
from jax.experimental.pallas import tpu as pltpu
from jax.experimental.pallas import tpu_sc as plsc

LAMDA = 0.001

NC = 2
NS = 16
NW = NC * NS
L = 16

B = 16384
D = 64
BPW = B // NW
CH = 64
NCHUNK = BPW // CH


def _sc_body(u_h, i_h, pu_h, pi_h, nu_h, ni_h, r_h, nr_h, wu, wi, wr,
             xhat_o, sq_o,
             u_v, i_v, pu_v, pi_v, nu_v, ni_v, r_v, nr_v,
             ri_v, pri_v, nri_v,
             bA0, bA1, bA2, bA3, bA4, bA5, bA6, bA7, bA8,
             bB0, bB1, bB2, bB3, bB4, bB5, bB6, bB7, bB8,
             xhat_v, squ, sqi, sqpu, sqpi, sqnu, sqni, sqr, sqnr,
             semA, semB):
    user_num = wu.shape[0]
    wid = lax.axis_index("s") * NC + lax.axis_index("c")
    base = pl.multiple_of(wid * BPW, BPW)

    pltpu.sync_copy(u_h.at[pl.ds(base, BPW)], u_v)
    pltpu.sync_copy(i_h.at[pl.ds(base, BPW)], i_v)
    pltpu.sync_copy(pu_h.at[pl.ds(base, BPW)], pu_v)
    pltpu.sync_copy(pi_h.at[pl.ds(base, BPW)], pi_v)
    pltpu.sync_copy(nu_h.at[pl.ds(base, BPW)], nu_v)
    pltpu.sync_copy(ni_h.at[pl.ds(base, BPW)], ni_v)
    pltpu.sync_copy(r_h.at[pl.ds(base, BPW)], r_v)
    pltpu.sync_copy(nr_h.at[pl.ds(base, BPW)], nr_v)

    def idx_body(k, _):
        s = pl.ds(pl.multiple_of(k * L, L), L)
        rv = r_v[s]
        ri_v[s] = u_v[s] + rv * user_num
        pri_v[s] = pu_v[s] + rv * user_num
        nri_v[s] = nu_v[s] + nr_v[s] * user_num
        return 0

    lax.fori_loop(0, BPW // L, idx_body, 0)

    rows0 = lax.iota(jnp.int32, L)
    zero = jnp.zeros((L,), jnp.float32)

    def hsum(v):
        for sh in (8, 4, 2, 1):
            perm = jnp.bitwise_xor(rows0, sh)
            v = v + jnp.take_along_axis(v, perm, axis=0,
                                        mode="promise_in_bounds")
        return v

    tabs = [wu, wi, wu, wi, wu, wi, wr, wr, wr]
    idxs = [u_v, i_v, pu_v, pi_v, nu_v, ni_v, ri_v, pri_v, nri_v]
    sets = [[bA0, bA1, bA2, bA3, bA4, bA5, bA6, bA7, bA8],
            [bB0, bB1, bB2, bB3, bB4, bB5, bB6, bB7, bB8]]
    sems = [semA, semB]

    def issue(c, bufs, sem):
        s = pl.ds(c * CH, CH)
        return [pltpu.async_copy(t.at[ix.at[s]], b, sem)
                for t, ix, b in zip(tabs, idxs, bufs)]

    def compute(c, bufs):
        bu, bi, bpu, bpi, bnu, bni, br, bpr, bnr = bufs

        def group_body(g, _, c=c):
            def row_body(rr, acc):
                xh, au, ai, apu, api, anu, ani, ar, anr = acc
                r = g * L + rr
                part = [zero] * 9
                for k in range(D // L):
                    sk = pl.ds(k * L, L)
                    uv = bu[r, sk]
                    iv = bi[r, sk]
                    puv = bpu[r, sk]
                    piv = bpi[r, sk]
                    nuv = bnu[r, sk]
                    niv = bni[r, sk]
                    rv = br[r, sk]
                    prv = bpr[r, sk]
                    nrv = bnr[r, sk]
                    gv = uv + rv + iv
                    gp = puv + prv + piv
                    gn = nuv + nrv + niv
                    part = [part[0] + gv * (gp - gn), part[1] + uv * uv,
                            part[2] + iv * iv, part[3] + puv * puv,
                            part[4] + piv * piv, part[5] + nuv * nuv,
                            part[6] + niv * niv, part[7] + rv * rv,
                            part[8] + nrv * nrv]
                lane = rows0 == rr
                xh = jnp.where(lane, hsum(part[0]), xh)
                au = jnp.where(lane, hsum(part[1]), au)
                ai = jnp.where(lane, hsum(part[2]), ai)
                apu = jnp.where(lane, hsum(part[3]), apu)
                api = jnp.where(lane, hsum(part[4]), api)
                anu = jnp.where(lane, hsum(part[5]), anu)
                ani = jnp.where(lane, hsum(part[6]), ani)
                ar = jnp.where(lane, hsum(part[7]), ar)
                anr = jnp.where(lane, hsum(part[8]), anr)
                return (xh, au, ai, apu, api, anu, ani, ar, anr)

            xh, au, ai, apu, api, anu, ani, ar, anr = lax.fori_loop(
                0, L, row_body, (zero,) * 9)
            so = pl.ds(pl.multiple_of(c * CH + g * L, L), L)
            xhat_v[so] = xh
            squ[so] = au
            sqi[so] = ai
            sqpu[so] = apu
            sqpi[so] = api
            sqnu[so] = anu
            sqni[so] = ani
            sqr[so] = ar
            sqnr[so] = anr
            return 0

        lax.fori_loop(0, CH // L, group_body, 0)

    pend = [None, None]
    for c in range(NCHUNK + 1):
        if c < NCHUNK:
            pend[c % 2] = issue(c, sets[c % 2], sems[c % 2])
        if c >= 1:
            for cp in pend[(c + 1) % 2]:
                cp.wait()
            compute(c - 1, sets[(c + 1) % 2])

    pltpu.sync_copy(xhat_v, xhat_o.at[pl.ds(base, BPW)])
    pltpu.sync_copy(squ, sq_o.at[0, pl.ds(base, BPW)])
    pltpu.sync_copy(sqi, sq_o.at[1, pl.ds(base, BPW)])
    pltpu.sync_copy(sqpu, sq_o.at[2, pl.ds(base, BPW)])
    pltpu.sync_copy(sqpi, sq_o.at[3, pl.ds(base, BPW)])
    pltpu.sync_copy(sqnu, sq_o.at[4, pl.ds(base, BPW)])
    pltpu.sync_copy(sqni, sq_o.at[5, pl.ds(base, BPW)])
    pltpu.sync_copy(sqr, sq_o.at[6, pl.ds(base, BPW)])
    pltpu.sync_copy(sqnr, sq_o.at[7, pl.ds(base, BPW)])


_sc_call = functools.partial(
    pl.kernel,
    out_type=(jax.ShapeDtypeStruct((B,), jnp.float32),
              jax.ShapeDtypeStruct((8, B), jnp.float32)),
    mesh=plsc.VectorSubcoreMesh(core_axis_name="c", subcore_axis_name="s",
                                num_cores=NC, num_subcores=NS),
    scratch_types=(
        [pltpu.VMEM((BPW,), jnp.int32)] * 11
        + [pltpu.VMEM((CH, D), jnp.float32)] * 18
        + [pltpu.VMEM((BPW,), jnp.float32)] * 9
        + [pltpu.SemaphoreType.DMA] * 2
    ),
    compiler_params=pltpu.CompilerParams(use_tc_tiling_on_sc=False),
)(_sc_body)


def _fin_body(x_ref, s_ref, loss_ref, reg_ref):
    x = x_ref[...]
    p = 1.0 / (1.0 + jnp.exp(-x))
    loss_ref[0, 0] = -jnp.sum(jnp.log(p))
    reg_ref[0, 0] = jnp.sum(jnp.sqrt(s_ref[...])) * LAMDA


_fin_call = pl.pallas_call(
    _fin_body,
    out_shape=(jax.ShapeDtypeStruct((1, 1), jnp.float32),
               jax.ShapeDtypeStruct((1, 1), jnp.float32)),
    out_specs=(pl.BlockSpec(memory_space=pltpu.SMEM),
               pl.BlockSpec(memory_space=pltpu.SMEM)),
)


def kernel(user_idx, item_idx, pos_user_idx, pos_item_idx, neg_user_idx,
           neg_item_idx, rel_idx, neg_rel_idx, W_user, W_item, W_rel):
    xhat, sq = _sc_call(user_idx.astype(jnp.int32), item_idx.astype(jnp.int32),
                        pos_user_idx.astype(jnp.int32),
                        pos_item_idx.astype(jnp.int32),
                        neg_user_idx.astype(jnp.int32),
                        neg_item_idx.astype(jnp.int32),
                        rel_idx.astype(jnp.int32),
                        neg_rel_idx.astype(jnp.int32),
                        W_user, W_item, W_rel)
    loss, reg = _fin_call(xhat.reshape(128, 128), sq.reshape(1024, 128))
    return (loss[0, 0], reg[0, 0])


# speedup vs baseline: 1.0927x; 1.0469x over previous
import functools

import jax
import jax.numpy as jnp
from jax import lax
from jax.experimental import pallas as pl
from jax.experimental.pallas import tpu as pltpu
from jax.experimental.pallas import tpu_sc as plsc

LAMDA = 0.001

NC = 2
NS = 16
NW = NC * NS
L = 16

B = 16384
D = 64
BPW = B // NW
CH = 64
NCHUNK = BPW // CH

_MESH = plsc.VectorSubcoreMesh(core_axis_name="c", subcore_axis_name="s",
                               num_cores=NC, num_subcores=NS)


def _hsum(v, rows0):
    for sh in (8, 4, 2, 1):
        perm = jnp.bitwise_xor(rows0, sh)
        v = v + jnp.take_along_axis(v, perm, axis=0,
                                    mode="promise_in_bounds")
    return v


def _stage(idx_hbm, idx_vmem, base):
    pltpu.sync_copy(idx_hbm.at[pl.ds(base, BPW)], idx_vmem)


def _a_body(u_h, i_h, pu_h, pi_h, nu_h, ni_h, wu, wi,
            a_o, dp_o, sq_o,
            u_v, i_v, pu_v, pi_v, nu_v, ni_v,
            bA0, bA1, bA2, bA3, bA4, bA5,
            bB0, bB1, bB2, bB3, bB4, bB5,
            av, dpv,
            squ, sqi, sqpu, sqpi, sqnu, sqni,
            semA, semB, semO):
    wid = lax.axis_index("s") * NC + lax.axis_index("c")
    base = pl.multiple_of(wid * BPW, BPW)
    for h, v in ((u_h, u_v), (i_h, i_v), (pu_h, pu_v), (pi_h, pi_v),
                 (nu_h, nu_v), (ni_h, ni_v)):
        _stage(h, v, base)

    rows0 = lax.iota(jnp.int32, L)
    zero = jnp.zeros((L,), jnp.float32)
    tabs = [wu, wi, wu, wi, wu, wi]
    idxs = [u_v, i_v, pu_v, pi_v, nu_v, ni_v]
    sets = [[bA0, bA1, bA2, bA3, bA4, bA5], [bB0, bB1, bB2, bB3, bB4, bB5]]
    sems = [semA, semB]

    def issue(c, bufs, sem):
        s = pl.ds(c * CH, CH)
        return [pltpu.async_copy(t.at[ix.at[s]], bb, sem)
                for t, ix, bb in zip(tabs, idxs, bufs)]

    def compute(c, bufs):
        bu, bi, bpu, bpi, bnu, bni = bufs

        def group_body(g, _, c=c):
            def row_body(rr, acc):
                au, ai, apu, api, anu, ani = acc
                r = g * L + rr
                part = [zero] * 6
                for k in range(D // L):
                    sk = pl.ds(k * L, L)
                    uv = bu[r, sk]
                    iv = bi[r, sk]
                    puv = bpu[r, sk]
                    piv = bpi[r, sk]
                    nuv = bnu[r, sk]
                    niv = bni[r, sk]
                    av[r, sk] = uv + iv
                    dpv[r, sk] = (puv + piv) - (nuv + niv)
                    part = [part[0] + uv * uv, part[1] + iv * iv,
                            part[2] + puv * puv, part[3] + piv * piv,
                            part[4] + nuv * nuv, part[5] + niv * niv]
                lane = rows0 == rr
                au = jnp.where(lane, _hsum(part[0], rows0), au)
                ai = jnp.where(lane, _hsum(part[1], rows0), ai)
                apu = jnp.where(lane, _hsum(part[2], rows0), apu)
                api = jnp.where(lane, _hsum(part[3], rows0), api)
                anu = jnp.where(lane, _hsum(part[4], rows0), anu)
                ani = jnp.where(lane, _hsum(part[5], rows0), ani)
                return (au, ai, apu, api, anu, ani)

            au, ai, apu, api, anu, ani = lax.fori_loop(
                0, L, row_body, (zero,) * 6)
            so = pl.ds(pl.multiple_of(c * CH + g * L, L), L)
            squ[so] = au
            sqi[so] = ai
            sqpu[so] = apu
            sqpi[so] = api
            sqnu[so] = anu
            sqni[so] = ani
            return 0

        lax.fori_loop(0, CH // L, group_body, 0)
        s = pl.ds(pl.multiple_of(base + c * CH, CH), CH)
        cp1 = pltpu.async_copy(av, a_o.at[s], semO)
        cp2 = pltpu.async_copy(dpv, dp_o.at[s], semO)
        return cp1, cp2

    pend = [None, None]
    out_pend = []
    for c in range(NCHUNK + 1):
        if c < NCHUNK:
            pend[c % 2] = issue(c, sets[c % 2], sems[c % 2])
        if c >= 1:
            for cp in pend[(c + 1) % 2]:
                cp.wait()
            for cp in out_pend:
                cp.wait()
            out_pend = list(compute(c - 1, sets[(c + 1) % 2]))
    for cp in out_pend:
        cp.wait()

    for t, v in enumerate((squ, sqi, sqpu, sqpi, sqnu, sqni)):
        pltpu.sync_copy(v, sq_o.at[t, pl.ds(base, BPW)])


_a_call = functools.partial(
    pl.kernel,
    out_type=(jax.ShapeDtypeStruct((B, D), jnp.float32),
              jax.ShapeDtypeStruct((B, D), jnp.float32),
              jax.ShapeDtypeStruct((6, B), jnp.float32)),
    mesh=_MESH,
    scratch_types=(
        [pltpu.VMEM((BPW,), jnp.int32)] * 6
        + [pltpu.VMEM((CH, D), jnp.float32)] * 12
        + [pltpu.VMEM((CH, D), jnp.float32)] * 2
        + [pltpu.VMEM((BPW,), jnp.float32)] * 6
        + [pltpu.SemaphoreType.DMA] * 3
    ),
    compiler_params=pltpu.CompilerParams(use_tc_tiling_on_sc=False),
)(_a_body)


def _b_body(u_h, pu_h, nu_h, r_h, nr_h, wr, a_h, dp_h,
            xhat_o, sq_o,
            u_v, pu_v, nu_v, r_v, nr_v,
            ri_v, pri_v, nri_v,
            bA0, bA1, bA2, bA3, bA4,
            bB0, bB1, bB2, bB3, bB4,
            xhat_v, sqr, sqnr,
            semA, semB):
    user_num = wr.shape[0] // 3
    wid = lax.axis_index("s") * NC + lax.axis_index("c")
    base = pl.multiple_of(wid * BPW, BPW)
    for h, v in ((u_h, u_v), (pu_h, pu_v), (nu_h, nu_v), (r_h, r_v),
                 (nr_h, nr_v)):
        _stage(h, v, base)

    def idx_body(k, _):
        s = pl.ds(pl.multiple_of(k * L, L), L)
        rv = r_v[s]
        ri_v[s] = u_v[s] + rv * user_num
        pri_v[s] = pu_v[s] + rv * user_num
        nri_v[s] = nu_v[s] + nr_v[s] * user_num
        return 0

    lax.fori_loop(0, BPW // L, idx_body, 0)

    rows0 = lax.iota(jnp.int32, L)
    zero = jnp.zeros((L,), jnp.float32)
    sets = [[bA0, bA1, bA2, bA3, bA4], [bB0, bB1, bB2, bB3, bB4]]
    sems = [semA, semB]

    def issue(c, bufs, sem):
        s = pl.ds(c * CH, CH)
        so = pl.ds(pl.multiple_of(base + c * CH, CH), CH)
        br, bpr, bnr, ba, bdp = bufs
        return [pltpu.async_copy(wr.at[ri_v.at[s]], br, sem),
                pltpu.async_copy(wr.at[pri_v.at[s]], bpr, sem),
                pltpu.async_copy(wr.at[nri_v.at[s]], bnr, sem),
                pltpu.async_copy(a_h.at[so], ba, sem),
                pltpu.async_copy(dp_h.at[so], bdp, sem)]

    def compute(c, bufs):
        br, bpr, bnr, ba, bdp = bufs

        def group_body(g, _, c=c):
            def row_body(rr, acc):
                xh, ar, anr = acc
                r = g * L + rr
                pxh = zero
                par = zero
                panr = zero
                for k in range(D // L):
                    sk = pl.ds(k * L, L)
                    rv = br[r, sk]
                    prv = bpr[r, sk]
                    nrv = bnr[r, sk]
                    avv = ba[r, sk]
                    dpvv = bdp[r, sk]
                    pxh = pxh + (avv + rv) * (dpvv + (prv - nrv))
                    par = par + rv * rv
                    panr = panr + nrv * nrv
                lane = rows0 == rr
                xh = jnp.where(lane, _hsum(pxh, rows0), xh)
                ar = jnp.where(lane, _hsum(par, rows0), ar)
                anr = jnp.where(lane, _hsum(panr, rows0), anr)
                return (xh, ar, anr)

            xh, ar, anr = lax.fori_loop(0, L, row_body, (zero,) * 3)
            so = pl.ds(pl.multiple_of(c * CH + g * L, L), L)
            xhat_v[so] = xh
            sqr[so] = ar
            sqnr[so] = anr
            return 0

        lax.fori_loop(0, CH // L, group_body, 0)

    pend = [None, None]
    for c in range(NCHUNK + 1):
        if c < NCHUNK:
            pend[c % 2] = issue(c, sets[c % 2], sems[c % 2])
        if c >= 1:
            for cp in pend[(c + 1) % 2]:
                cp.wait()
            compute(c - 1, sets[(c + 1) % 2])

    pltpu.sync_copy(xhat_v, xhat_o.at[pl.ds(base, BPW)])
    pltpu.sync_copy(sqr, sq_o.at[0, pl.ds(base, BPW)])
    pltpu.sync_copy(sqnr, sq_o.at[1, pl.ds(base, BPW)])


_b_call = functools.partial(
    pl.kernel,
    out_type=(jax.ShapeDtypeStruct((B,), jnp.float32),
              jax.ShapeDtypeStruct((2, B), jnp.float32)),
    mesh=_MESH,
    scratch_types=(
        [pltpu.VMEM((BPW,), jnp.int32)] * 8
        + [pltpu.VMEM((CH, D), jnp.float32)] * 10
        + [pltpu.VMEM((BPW,), jnp.float32)] * 3
        + [pltpu.SemaphoreType.DMA] * 2
    ),
    compiler_params=pltpu.CompilerParams(use_tc_tiling_on_sc=False),
)(_b_body)


def _fin_body(x_ref, sa_ref, sb_ref, loss_ref, reg_ref):
    x = x_ref[...]
    p = 1.0 / (1.0 + jnp.exp(-x))
    loss_ref[0, 0] = -jnp.sum(jnp.log(p))
    reg_ref[0, 0] = (jnp.sum(jnp.sqrt(sa_ref[...]))
                     + jnp.sum(jnp.sqrt(sb_ref[...]))) * LAMDA


_fin_call = pl.pallas_call(
    _fin_body,
    out_shape=(jax.ShapeDtypeStruct((1, 1), jnp.float32),
               jax.ShapeDtypeStruct((1, 1), jnp.float32)),
    out_specs=(pl.BlockSpec(memory_space=pltpu.SMEM),
               pl.BlockSpec(memory_space=pltpu.SMEM)),
)


def kernel(user_idx, item_idx, pos_user_idx, pos_item_idx, neg_user_idx,
           neg_item_idx, rel_idx, neg_rel_idx, W_user, W_item, W_rel):
    u = user_idx.astype(jnp.int32)
    i = item_idx.astype(jnp.int32)
    pu = pos_user_idx.astype(jnp.int32)
    pi = pos_item_idx.astype(jnp.int32)
    nu = neg_user_idx.astype(jnp.int32)
    ni = neg_item_idx.astype(jnp.int32)
    r = rel_idx.astype(jnp.int32)
    nr = neg_rel_idx.astype(jnp.int32)
    a, dp, sqa = _a_call(u, i, pu, pi, nu, ni, W_user, W_item)
    xhat, sqb = _b_call(u, pu, nu, r, nr, W_rel, a, dp)
    loss, reg = _fin_call(xhat.reshape(128, 128), sqa.reshape(768, 128),
                          sqb.reshape(256, 128))
    return (loss[0, 0], reg[0, 0])


# speedup vs baseline: 1.0987x; 1.0055x over previous
import functools

import jax
import jax.numpy as jnp
from jax import lax
from jax.experimental import pallas as pl
from jax.experimental.pallas import tpu as pltpu
from jax.experimental.pallas import tpu_sc as plsc

LAMDA = 0.001

NC = 2
NS = 16
NW = NC * NS
L = 16

B = 16384
D = 64
BPW = B // NW
CH = 64
NCHUNK = BPW // CH
CHB = 128
NCHUNKB = BPW // CHB

_MESH = plsc.VectorSubcoreMesh(core_axis_name="c", subcore_axis_name="s",
                               num_cores=NC, num_subcores=NS)


def _hsum(v, rows0):
    for sh in (8, 4, 2, 1):
        perm = jnp.bitwise_xor(rows0, sh)
        v = v + jnp.take_along_axis(v, perm, axis=0,
                                    mode="promise_in_bounds")
    return v


def _stage(idx_hbm, idx_vmem, base):
    pltpu.sync_copy(idx_hbm.at[pl.ds(base, BPW)], idx_vmem)


def _a_body(u_h, i_h, pu_h, pi_h, nu_h, ni_h, wu, wi,
            a_o, dp_o, sq_o,
            u_v, i_v, pu_v, pi_v, nu_v, ni_v,
            bA0, bA1, bA2, bA3, bA4, bA5,
            bB0, bB1, bB2, bB3, bB4, bB5,
            av, dpv,
            squ, sqi, sqpu, sqpi, sqnu, sqni,
            semA, semB, semO):
    wid = lax.axis_index("s") * NC + lax.axis_index("c")
    base = pl.multiple_of(wid * BPW, BPW)
    for h, v in ((u_h, u_v), (i_h, i_v), (pu_h, pu_v), (pi_h, pi_v),
                 (nu_h, nu_v), (ni_h, ni_v)):
        _stage(h, v, base)

    rows0 = lax.iota(jnp.int32, L)
    zero = jnp.zeros((L,), jnp.float32)
    tabs = [wu, wi, wu, wi, wu, wi]
    idxs = [u_v, i_v, pu_v, pi_v, nu_v, ni_v]
    sets = [[bA0, bA1, bA2, bA3, bA4, bA5], [bB0, bB1, bB2, bB3, bB4, bB5]]
    sems = [semA, semB]

    def issue(c, bufs, sem):
        s = pl.ds(c * CH, CH)
        return [pltpu.async_copy(t.at[ix.at[s]], bb, sem)
                for t, ix, bb in zip(tabs, idxs, bufs)]

    def compute(c, bufs):
        bu, bi, bpu, bpi, bnu, bni = bufs

        def group_body(g, _, c=c):
            def row_body(rr, acc):
                au, ai, apu, api, anu, ani = acc
                r = g * L + rr
                part = [zero] * 6
                for k in range(D // L):
                    sk = pl.ds(k * L, L)
                    uv = bu[r, sk]
                    iv = bi[r, sk]
                    puv = bpu[r, sk]
                    piv = bpi[r, sk]
                    nuv = bnu[r, sk]
                    niv = bni[r, sk]
                    av[r, sk] = uv + iv
                    dpv[r, sk] = (puv + piv) - (nuv + niv)
                    part = [part[0] + uv * uv, part[1] + iv * iv,
                            part[2] + puv * puv, part[3] + piv * piv,
                            part[4] + nuv * nuv, part[5] + niv * niv]
                lane = rows0 == rr
                au = jnp.where(lane, _hsum(part[0], rows0), au)
                ai = jnp.where(lane, _hsum(part[1], rows0), ai)
                apu = jnp.where(lane, _hsum(part[2], rows0), apu)
                api = jnp.where(lane, _hsum(part[3], rows0), api)
                anu = jnp.where(lane, _hsum(part[4], rows0), anu)
                ani = jnp.where(lane, _hsum(part[5], rows0), ani)
                return (au, ai, apu, api, anu, ani)

            au, ai, apu, api, anu, ani = lax.fori_loop(
                0, L, row_body, (zero,) * 6)
            so = pl.ds(pl.multiple_of(c * CH + g * L, L), L)
            squ[so] = au
            sqi[so] = ai
            sqpu[so] = apu
            sqpi[so] = api
            sqnu[so] = anu
            sqni[so] = ani
            return 0

        lax.fori_loop(0, CH // L, group_body, 0)
        s = pl.ds(pl.multiple_of(base + c * CH, CH), CH)
        cp1 = pltpu.async_copy(av, a_o.at[s], semO)
        cp2 = pltpu.async_copy(dpv, dp_o.at[s], semO)
        return cp1, cp2

    pend = [None, None]
    out_pend = []
    for c in range(NCHUNK + 1):
        if c < NCHUNK:
            pend[c % 2] = issue(c, sets[c % 2], sems[c % 2])
        if c >= 1:
            for cp in pend[(c + 1) % 2]:
                cp.wait()
            for cp in out_pend:
                cp.wait()
            out_pend = list(compute(c - 1, sets[(c + 1) % 2]))
    for cp in out_pend:
        cp.wait()

    for t, v in enumerate((squ, sqi, sqpu, sqpi, sqnu, sqni)):
        pltpu.sync_copy(v, sq_o.at[t, pl.ds(base, BPW)])


_a_call = functools.partial(
    pl.kernel,
    out_type=(jax.ShapeDtypeStruct((B, D), jnp.float32),
              jax.ShapeDtypeStruct((B, D), jnp.float32),
              jax.ShapeDtypeStruct((6, B), jnp.float32)),
    mesh=_MESH,
    scratch_types=(
        [pltpu.VMEM((BPW,), jnp.int32)] * 6
        + [pltpu.VMEM((CH, D), jnp.float32)] * 12
        + [pltpu.VMEM((CH, D), jnp.float32)] * 2
        + [pltpu.VMEM((BPW,), jnp.float32)] * 6
        + [pltpu.SemaphoreType.DMA] * 3
    ),
    compiler_params=pltpu.CompilerParams(use_tc_tiling_on_sc=False),
)(_a_body)


def _b_body(u_h, pu_h, nu_h, r_h, nr_h, wr, a_h, dp_h,
            xhat_o, sq_o,
            u_v, pu_v, nu_v, r_v, nr_v,
            ri_v, pri_v, nri_v,
            bA0, bA1, bA2, bA3, bA4,
            bB0, bB1, bB2, bB3, bB4,
            xhat_v, sqr, sqnr,
            semA, semB):
    user_num = wr.shape[0] // 3
    wid = lax.axis_index("s") * NC + lax.axis_index("c")
    base = pl.multiple_of(wid * BPW, BPW)
    for h, v in ((u_h, u_v), (pu_h, pu_v), (nu_h, nu_v), (r_h, r_v),
                 (nr_h, nr_v)):
        _stage(h, v, base)

    def idx_body(k, _):
        s = pl.ds(pl.multiple_of(k * L, L), L)
        rv = r_v[s]
        ri_v[s] = u_v[s] + rv * user_num
        pri_v[s] = pu_v[s] + rv * user_num
        nri_v[s] = nu_v[s] + nr_v[s] * user_num
        return 0

    lax.fori_loop(0, BPW // L, idx_body, 0)

    rows0 = lax.iota(jnp.int32, L)
    zero = jnp.zeros((L,), jnp.float32)
    sets = [[bA0, bA1, bA2, bA3, bA4], [bB0, bB1, bB2, bB3, bB4]]
    sems = [semA, semB]

    def issue(c, bufs, sem):
        s = pl.ds(c * CHB, CHB)
        so = pl.ds(pl.multiple_of(base + c * CHB, CHB), CHB)
        br, bpr, bnr, ba, bdp = bufs
        return [pltpu.async_copy(wr.at[ri_v.at[s]], br, sem),
                pltpu.async_copy(wr.at[pri_v.at[s]], bpr, sem),
                pltpu.async_copy(wr.at[nri_v.at[s]], bnr, sem),
                pltpu.async_copy(a_h.at[so], ba, sem),
                pltpu.async_copy(dp_h.at[so], bdp, sem)]

    def compute(c, bufs):
        br, bpr, bnr, ba, bdp = bufs

        def group_body(g, _, c=c):
            def row_body(rr, acc):
                xh, ar, anr = acc
                r = g * L + rr
                pxh = zero
                par = zero
                panr = zero
                for k in range(D // L):
                    sk = pl.ds(k * L, L)
                    rv = br[r, sk]
                    prv = bpr[r, sk]
                    nrv = bnr[r, sk]
                    avv = ba[r, sk]
                    dpvv = bdp[r, sk]
                    pxh = pxh + (avv + rv) * (dpvv + (prv - nrv))
                    par = par + rv * rv
                    panr = panr + nrv * nrv
                lane = rows0 == rr
                xh = jnp.where(lane, _hsum(pxh, rows0), xh)
                ar = jnp.where(lane, _hsum(par, rows0), ar)
                anr = jnp.where(lane, _hsum(panr, rows0), anr)
                return (xh, ar, anr)

            xh, ar, anr = lax.fori_loop(0, L, row_body, (zero,) * 3)
            so = pl.ds(pl.multiple_of(c * CHB + g * L, L), L)
            xhat_v[so] = xh
            sqr[so] = ar
            sqnr[so] = anr
            return 0

        lax.fori_loop(0, CHB // L, group_body, 0)

    pend = [None, None]
    for c in range(NCHUNKB + 1):
        if c < NCHUNKB:
            pend[c % 2] = issue(c, sets[c % 2], sems[c % 2])
        if c >= 1:
            for cp in pend[(c + 1) % 2]:
                cp.wait()
            compute(c - 1, sets[(c + 1) % 2])

    pltpu.sync_copy(xhat_v, xhat_o.at[pl.ds(base, BPW)])
    pltpu.sync_copy(sqr, sq_o.at[0, pl.ds(base, BPW)])
    pltpu.sync_copy(sqnr, sq_o.at[1, pl.ds(base, BPW)])


_b_call = functools.partial(
    pl.kernel,
    out_type=(jax.ShapeDtypeStruct((B,), jnp.float32),
              jax.ShapeDtypeStruct((2, B), jnp.float32)),
    mesh=_MESH,
    scratch_types=(
        [pltpu.VMEM((BPW,), jnp.int32)] * 8
        + [pltpu.VMEM((CHB, D), jnp.float32)] * 10
        + [pltpu.VMEM((BPW,), jnp.float32)] * 3
        + [pltpu.SemaphoreType.DMA] * 2
    ),
    compiler_params=pltpu.CompilerParams(use_tc_tiling_on_sc=False),
)(_b_body)


def _fin_body(x_ref, sa_ref, sb_ref, loss_ref, reg_ref):
    x = x_ref[...]
    p = 1.0 / (1.0 + jnp.exp(-x))
    loss_ref[0, 0] = -jnp.sum(jnp.log(p))
    reg_ref[0, 0] = (jnp.sum(jnp.sqrt(sa_ref[...]))
                     + jnp.sum(jnp.sqrt(sb_ref[...]))) * LAMDA


_fin_call = pl.pallas_call(
    _fin_body,
    out_shape=(jax.ShapeDtypeStruct((1, 1), jnp.float32),
               jax.ShapeDtypeStruct((1, 1), jnp.float32)),
    out_specs=(pl.BlockSpec(memory_space=pltpu.SMEM),
               pl.BlockSpec(memory_space=pltpu.SMEM)),
)


def kernel(user_idx, item_idx, pos_user_idx, pos_item_idx, neg_user_idx,
           neg_item_idx, rel_idx, neg_rel_idx, W_user, W_item, W_rel):
    u = user_idx.astype(jnp.int32)
    i = item_idx.astype(jnp.int32)
    pu = pos_user_idx.astype(jnp.int32)
    pi = pos_item_idx.astype(jnp.int32)
    nu = neg_user_idx.astype(jnp.int32)
    ni = neg_item_idx.astype(jnp.int32)
    r = rel_idx.astype(jnp.int32)
    nr = neg_rel_idx.astype(jnp.int32)
    a, dp, sqa = _a_call(u, i, pu, pi, nu, ni, W_user, W_item)
    xhat, sqb = _b_call(u, pu, nu, r, nr, W_rel, a, dp)
    loss, reg = _fin_call(xhat.reshape(128, 128), sqa.reshape(768, 128),
                          sqb.reshape(256, 128))
    return (loss[0, 0], reg[0, 0])
